# baseline (speedup 1.0000x reference)
import jax
import jax.numpy as jnp
from jax import lax
from jax.experimental import pallas as pl
from jax.experimental.pallas import tpu as pltpu
from jax.experimental.pallas import tpu_sc as plsc

BATCH = 4096
DIM = 2048
NC = 2
NS = 16
NW = NC * NS
ROWS_PER_W = BATCH // NW
RB = 4
NBLK = ROWS_PER_W // RB
NCH = DIM // 16
NBUF_IN = 8
NBUF = 6


def _sc_body(x_hbm, idx_hbm, z_hbm, ld_hbm,
             idx_v, in0, in1, in2, in3, in4, in5, in6, in7,
             out0, out1, out2, out3, out4, out5, zero_v,
             si0, si1, si2, si3, si4, si5, si6, si7,
             so0, so1, so2, so3, so4, so5, sem_ld):
    wid = lax.axis_index("s") * NC + lax.axis_index("c")
    row0 = wid * ROWS_PER_W
    ins = (in0, in1, in2, in3, in4, in5, in6, in7)
    outs = (out0, out1, out2, out3, out4, out5)
    sems_in = (si0, si1, si2, si3, si4, si5, si6, si7)
    sems_out = (so0, so1, so2, so3, so4, so5)

    def start_in(b):
        return pltpu.async_copy(
            x_hbm.at[pl.ds(row0 + b * RB, RB)], ins[b % NBUF_IN],
            sems_in[b % NBUF_IN])

    h_in = {b: start_in(b) for b in range(NBUF_IN)}
    h_out = {}

    pltpu.sync_copy(idx_hbm, idx_v)

    for i in range(ROWS_PER_W // 16):
        zero_v[pl.ds(16 * i, 16)] = jnp.zeros((16,), jnp.float32)
    h_ld = pltpu.async_copy(zero_v, ld_hbm.at[pl.ds(row0, ROWS_PER_W)], sem_ld)
    for b in range(NBLK):
        cur = b % NBUF
        h_in[b].wait()
        if b >= NBUF:
            h_out[b - NBUF].wait()
        iv = ins[b % NBUF_IN]
        ov = outs[cur]

        @plsc.parallel_loop(0, NCH, 1, unroll=4)
        def chunk_body(j):
            idxv = idx_v[pl.ds(16 * j, 16)]
            for r in range(RB):
                rows = jnp.full((16,), r, jnp.int32)
                val = plsc.load_gather(iv, [rows, idxv])
                ov[r, pl.ds(16 * j, 16)] = val

        h_out[b] = pltpu.async_copy(
            ov, z_hbm.at[pl.ds(row0 + b * RB, RB)], sems_out[cur])
        if b + NBUF_IN < NBLK:
            h_in[b + NBUF_IN] = start_in(b + NBUF_IN)

    for b in range(NBLK - NBUF, NBLK):
        h_out[b].wait()
    h_ld.wait()


@jax.jit
def _sc_call(x, index):
    mesh = plsc.VectorSubcoreMesh(core_axis_name="c", subcore_axis_name="s")
    fn = pl.kernel(
        _sc_body,
        out_type=[
            jax.ShapeDtypeStruct((BATCH, DIM), jnp.float32),
            jax.ShapeDtypeStruct((BATCH,), jnp.float32),
        ],
        mesh=mesh,
        compiler_params=pltpu.CompilerParams(needs_layout_passes=False),
        scratch_types=[
            pltpu.VMEM((DIM,), jnp.int32),
            pltpu.VMEM((RB, DIM), jnp.float32),
            pltpu.VMEM((RB, DIM), jnp.float32),
            pltpu.VMEM((RB, DIM), jnp.float32),
            pltpu.VMEM((RB, DIM), jnp.float32),
            pltpu.VMEM((RB, DIM), jnp.float32),
            pltpu.VMEM((RB, DIM), jnp.float32),
            pltpu.VMEM((RB, DIM), jnp.float32),
            pltpu.VMEM((RB, DIM), jnp.float32),
            pltpu.VMEM((RB, DIM), jnp.float32),
            pltpu.VMEM((RB, DIM), jnp.float32),
            pltpu.VMEM((RB, DIM), jnp.float32),
            pltpu.VMEM((RB, DIM), jnp.float32),
            pltpu.VMEM((RB, DIM), jnp.float32),
            pltpu.VMEM((RB, DIM), jnp.float32),
            pltpu.VMEM((ROWS_PER_W,), jnp.float32),
            pltpu.SemaphoreType.DMA,
            pltpu.SemaphoreType.DMA,
            pltpu.SemaphoreType.DMA,
            pltpu.SemaphoreType.DMA,
            pltpu.SemaphoreType.DMA,
            pltpu.SemaphoreType.DMA,
            pltpu.SemaphoreType.DMA,
            pltpu.SemaphoreType.DMA,
            pltpu.SemaphoreType.DMA,
            pltpu.SemaphoreType.DMA,
            pltpu.SemaphoreType.DMA,
            pltpu.SemaphoreType.DMA,
            pltpu.SemaphoreType.DMA,
            pltpu.SemaphoreType.DMA,
            pltpu.SemaphoreType.DMA,
        ],
    )
    return fn(x, index)


def kernel(x, index):
    z, log_det = _sc_call(x, index)
    return (z, log_det)


# speedup vs baseline: 1.0706x; 1.0706x over previous
import jax
import jax.numpy as jnp
from jax import lax
from jax.experimental import pallas as pl
from jax.experimental.pallas import tpu as pltpu
from jax.experimental.pallas import tpu_sc as plsc

BATCH = 4096
DIM = 2048
NC = 2
NS = 16
NW = NC * NS
ROWS_PER_W = BATCH // NW
RB = 8
NBLK = ROWS_PER_W // RB
NCH = DIM // 16
NBUF_IN = 4
NBUF = 3


def _sc_body(x_hbm, idx_hbm, z_hbm, ld_hbm,
             idx_v, in0, in1, in2, in3, out0, out1, out2, zero_v,
             si0, si1, si2, si3, so0, so1, so2, sem_ld):
    wid = lax.axis_index("s") * NC + lax.axis_index("c")
    row0 = wid * ROWS_PER_W
    ins = (in0, in1, in2, in3)
    outs = (out0, out1, out2)
    sems_in = (si0, si1, si2, si3)
    sems_out = (so0, so1, so2)

    def start_in(b):
        return pltpu.async_copy(
            x_hbm.at[pl.ds(row0 + b * RB, RB)], ins[b % NBUF_IN],
            sems_in[b % NBUF_IN])

    h_in = {b: start_in(b) for b in range(NBUF_IN)}
    h_out = {}

    pltpu.sync_copy(idx_hbm, idx_v)

    for i in range(ROWS_PER_W // 16):
        zero_v[pl.ds(16 * i, 16)] = jnp.zeros((16,), jnp.float32)
    h_ld = pltpu.async_copy(zero_v, ld_hbm.at[pl.ds(row0, ROWS_PER_W)], sem_ld)
    for b in range(NBLK):
        cur = b % NBUF
        h_in[b].wait()
        if b >= NBUF:
            h_out[b - NBUF].wait()
        iv = ins[b % NBUF_IN]
        ov = outs[cur]

        @plsc.parallel_loop(0, NCH, 1, unroll=4)
        def chunk_body(j):
            idxv = idx_v[pl.ds(16 * j, 16)]
            for r in range(RB):
                rows = jnp.full((16,), r, jnp.int32)
                val = plsc.load_gather(iv, [rows, idxv])
                ov[r, pl.ds(16 * j, 16)] = val

        h_out[b] = pltpu.async_copy(
            ov, z_hbm.at[pl.ds(row0 + b * RB, RB)], sems_out[cur])
        if b + NBUF_IN < NBLK:
            h_in[b + NBUF_IN] = start_in(b + NBUF_IN)

    for b in range(NBLK - NBUF, NBLK):
        h_out[b].wait()
    h_ld.wait()


@jax.jit
def _sc_call(x, index):
    mesh = plsc.VectorSubcoreMesh(core_axis_name="c", subcore_axis_name="s")
    fn = pl.kernel(
        _sc_body,
        out_type=[
            jax.ShapeDtypeStruct((BATCH, DIM), jnp.float32),
            jax.ShapeDtypeStruct((BATCH,), jnp.float32),
        ],
        mesh=mesh,
        compiler_params=pltpu.CompilerParams(needs_layout_passes=False),
        scratch_types=[
            pltpu.VMEM((DIM,), jnp.int32),
            pltpu.VMEM((RB, DIM), jnp.float32),
            pltpu.VMEM((RB, DIM), jnp.float32),
            pltpu.VMEM((RB, DIM), jnp.float32),
            pltpu.VMEM((RB, DIM), jnp.float32),
            pltpu.VMEM((RB, DIM), jnp.float32),
            pltpu.VMEM((RB, DIM), jnp.float32),
            pltpu.VMEM((RB, DIM), jnp.float32),
            pltpu.VMEM((ROWS_PER_W,), jnp.float32),
            pltpu.SemaphoreType.DMA,
            pltpu.SemaphoreType.DMA,
            pltpu.SemaphoreType.DMA,
            pltpu.SemaphoreType.DMA,
            pltpu.SemaphoreType.DMA,
            pltpu.SemaphoreType.DMA,
            pltpu.SemaphoreType.DMA,
            pltpu.SemaphoreType.DMA,
        ],
    )
    return fn(x, index)


def kernel(x, index):
    z, log_det = _sc_call(x, index)
    return (z, log_det)
